# Initial kernel scaffold; baseline (speedup 1.0000x reference)
import jax
import jax.numpy as jnp
from jax.experimental import pallas as pl


def kernel(x, router_w, router_b, W_up, b_up, W_down, b_down):
    raise NotImplementedError("write your pallas kernel here")


# speedup vs baseline: 1.4212x; 1.4212x over previous
import functools

import jax
import jax.numpy as jnp
from jax.experimental import pallas as pl
from jax.experimental.pallas import tpu as pltpu

_TILE_T = 512


def _moe_body(nt, e_count, t_total,
              x32_ref, xb_ref, rw_ref, rb_ref,
              wu_ref, bu_ref, wd_ref, bd_ref,
              out_ref, fsum_ref, psum_ref, aux_ref,
              wfull_ref):
    t = pl.program_id(0)
    e = pl.program_id(1)

    @pl.when(e == 0)
    def _router():
        xt = x32_ref[...]
        logits = jax.lax.dot_general(
            xt, rw_ref[...], (((1,), (1,)), ((), ())),
            precision=jax.lax.Precision.DEFAULT,
            preferred_element_type=jnp.float32) + rb_ref[...]
        tt, ee = logits.shape
        ids = jax.lax.broadcasted_iota(jnp.int32, (tt, ee), 1)
        m1 = jnp.max(logits, axis=1, keepdims=True)
        a1 = jnp.min(jnp.where(logits == m1, ids, ee), axis=1, keepdims=True)
        sel1 = ids == a1
        rest = jnp.where(sel1, -jnp.inf, logits)
        m2 = jnp.max(rest, axis=1, keepdims=True)
        a2 = jnp.min(jnp.where(rest == m2, ids, ee), axis=1, keepdims=True)
        sel = sel1 | (ids == a2)
        ez = jnp.where(sel, jnp.exp(logits - m1), 0.0)
        w = ez / jnp.sum(ez, axis=1, keepdims=True)
        wfull_ref[...] = w

        @pl.when(t == 0)
        def _():
            fsum_ref[...] = jnp.zeros_like(fsum_ref)
            psum_ref[...] = jnp.zeros_like(psum_ref)

        fsum_ref[...] += jnp.sum(sel.astype(jnp.float32), axis=0)[None, :]
        psum_ref[...] += jnp.sum(w, axis=0)[None, :]

    xb = xb_ref[...]
    h = jnp.dot(xb, wu_ref[0], preferred_element_type=jnp.float32)
    h = h + bu_ref[0]
    g = 0.5 * h * (1.0 + jax.lax.erf(h * 0.7071067811865476))
    y = jnp.dot(g.astype(jnp.bfloat16), wd_ref[0],
                preferred_element_type=jnp.float32)
    y = y + bd_ref[0]
    wall = wfull_ref[...]
    eids = jax.lax.broadcasted_iota(jnp.int32, wall.shape, 1)
    wcol = jnp.sum(jnp.where(eids == e, wall, 0.0), axis=1, keepdims=True)
    contrib = wcol * y

    @pl.when(e == 0)
    def _():
        out_ref[...] = contrib

    @pl.when(e != 0)
    def _():
        out_ref[...] += contrib

    @pl.when((t == nt - 1) & (e == e_count - 1))
    def _aux():
        f = fsum_ref[...] * (1.0 / t_total)
        p = psum_ref[...] * (1.0 / t_total)
        aux_ref[...] = e_count * jnp.sum(f * p, keepdims=True)


def kernel(x, router_w, router_b, W_up, b_up, W_down, b_down):
    B, N, D = x.shape
    E, _, H = W_up.shape
    T = B * N
    tile = min(_TILE_T, T)
    nt = T // tile

    x2 = x.reshape(T, D)
    xb = x2.astype(jnp.bfloat16)
    wub = W_up.astype(jnp.bfloat16)
    wdb = W_down.astype(jnp.bfloat16)

    grid = (nt, E)
    out, fsum, psum, aux = pl.pallas_call(
        functools.partial(_moe_body, nt, E, float(T)),
        grid=grid,
        in_specs=[
            pl.BlockSpec((tile, D), lambda t, e: (t, 0)),
            pl.BlockSpec((tile, D), lambda t, e: (t, 0)),
            pl.BlockSpec((E, D), lambda t, e: (0, 0)),
            pl.BlockSpec((1, E), lambda t, e: (0, 0)),
            pl.BlockSpec((1, D, H), lambda t, e: (e, 0, 0)),
            pl.BlockSpec((1, 1, H), lambda t, e: (e, 0, 0)),
            pl.BlockSpec((1, H, D), lambda t, e: (e, 0, 0)),
            pl.BlockSpec((1, 1, D), lambda t, e: (e, 0, 0)),
        ],
        out_specs=[
            pl.BlockSpec((tile, D), lambda t, e: (t, 0)),
            pl.BlockSpec((1, E), lambda t, e: (0, 0)),
            pl.BlockSpec((1, E), lambda t, e: (0, 0)),
            pl.BlockSpec((1, 1), lambda t, e: (0, 0)),
        ],
        out_shape=[
            jax.ShapeDtypeStruct((T, D), jnp.float32),
            jax.ShapeDtypeStruct((1, E), jnp.float32),
            jax.ShapeDtypeStruct((1, E), jnp.float32),
            jax.ShapeDtypeStruct((1, 1), jnp.float32),
        ],
        scratch_shapes=[pltpu.VMEM((tile, E), jnp.float32)],
    )(x2, xb, router_w, router_b.reshape(1, E), wub,
      b_up.reshape(E, 1, H), wdb, b_down.reshape(E, 1, D))

    return out.reshape(B, N, D), aux[0, 0]


# speedup vs baseline: 1.4708x; 1.0349x over previous
import functools

import jax
import jax.numpy as jnp
from jax.experimental import pallas as pl
from jax.experimental.pallas import tpu as pltpu

_TILE_T = 1024


def _moe_body(nt, e_count, t_total,
              xb_ref, rw_ref, rb_ref,
              wu_ref, bu_ref, wd_ref, bd_ref,
              out_ref, fsum_ref, psum_ref, aux_ref,
              wfull_ref):
    t = pl.program_id(0)
    e = pl.program_id(1)

    @pl.when(e == 0)
    def _router():
        xt = xb_ref[...]
        logits = jax.lax.dot_general(
            xt, rw_ref[...], (((1,), (1,)), ((), ())),
            precision=jax.lax.Precision.DEFAULT,
            preferred_element_type=jnp.float32) + rb_ref[...]
        tt, ee = logits.shape
        ids = jax.lax.broadcasted_iota(jnp.int32, (tt, ee), 1)
        m1 = jnp.max(logits, axis=1, keepdims=True)
        a1 = jnp.min(jnp.where(logits == m1, ids, ee), axis=1, keepdims=True)
        sel1 = ids == a1
        rest = jnp.where(sel1, -jnp.inf, logits)
        m2 = jnp.max(rest, axis=1, keepdims=True)
        a2 = jnp.min(jnp.where(rest == m2, ids, ee), axis=1, keepdims=True)
        sel = sel1 | (ids == a2)
        ez = jnp.where(sel, jnp.exp(logits - m1), 0.0)
        w = ez / jnp.sum(ez, axis=1, keepdims=True)
        wfull_ref[...] = w

        @pl.when(t == 0)
        def _():
            fsum_ref[...] = jnp.zeros_like(fsum_ref)
            psum_ref[...] = jnp.zeros_like(psum_ref)

        fsum_ref[...] += jnp.sum(sel.astype(jnp.float32), axis=0)[None, :]
        psum_ref[...] += jnp.sum(w, axis=0)[None, :]

    xb = xb_ref[...]
    h = jnp.dot(xb, wu_ref[0], preferred_element_type=jnp.float32)
    h = h + bu_ref[0]
    g = 0.5 * h * (1.0 + jax.lax.erf(h * 0.7071067811865476))
    y = jnp.dot(g.astype(jnp.bfloat16), wd_ref[0],
                preferred_element_type=jnp.float32)
    y = y + bd_ref[0]
    wall = wfull_ref[...]
    eids = jax.lax.broadcasted_iota(jnp.int32, wall.shape, 1)
    wcol = jnp.sum(jnp.where(eids == e, wall, 0.0), axis=1, keepdims=True)
    contrib = wcol * y

    @pl.when(e == 0)
    def _():
        out_ref[...] = contrib

    @pl.when(e != 0)
    def _():
        out_ref[...] += contrib

    @pl.when((t == nt - 1) & (e == e_count - 1))
    def _aux():
        f = fsum_ref[...] * (1.0 / t_total)
        p = psum_ref[...] * (1.0 / t_total)
        aux_ref[...] = e_count * jnp.sum(f * p, keepdims=True)


def kernel(x, router_w, router_b, W_up, b_up, W_down, b_down):
    B, N, D = x.shape
    E, _, H = W_up.shape
    T = B * N
    tile = min(_TILE_T, T)
    nt = T // tile

    x2 = x.reshape(T, D)
    xb = x2.astype(jnp.bfloat16)
    wub = W_up.astype(jnp.bfloat16)
    wdb = W_down.astype(jnp.bfloat16)

    grid = (nt, E)
    out, fsum, psum, aux = pl.pallas_call(
        functools.partial(_moe_body, nt, E, float(T)),
        grid=grid,
        in_specs=[
            pl.BlockSpec((tile, D), lambda t, e: (t, 0)),
            pl.BlockSpec((E, D), lambda t, e: (0, 0)),
            pl.BlockSpec((1, E), lambda t, e: (0, 0)),
            pl.BlockSpec((1, D, H), lambda t, e: (e, 0, 0)),
            pl.BlockSpec((1, 1, H), lambda t, e: (e, 0, 0)),
            pl.BlockSpec((1, H, D), lambda t, e: (e, 0, 0)),
            pl.BlockSpec((1, 1, D), lambda t, e: (e, 0, 0)),
        ],
        out_specs=[
            pl.BlockSpec((tile, D), lambda t, e: (t, 0)),
            pl.BlockSpec((1, E), lambda t, e: (0, 0)),
            pl.BlockSpec((1, E), lambda t, e: (0, 0)),
            pl.BlockSpec((1, 1), lambda t, e: (0, 0)),
        ],
        out_shape=[
            jax.ShapeDtypeStruct((T, D), jnp.float32),
            jax.ShapeDtypeStruct((1, E), jnp.float32),
            jax.ShapeDtypeStruct((1, E), jnp.float32),
            jax.ShapeDtypeStruct((1, 1), jnp.float32),
        ],
        scratch_shapes=[pltpu.VMEM((tile, E), jnp.float32)],
    )(xb, router_w.astype(jnp.bfloat16), router_b.reshape(1, E), wub,
      b_up.reshape(E, 1, H), wdb, b_down.reshape(E, 1, D))

    return out.reshape(B, N, D), aux[0, 0]
